# baseline (speedup 1.0000x reference)
import jax
import jax.numpy as jnp
from jax.experimental import pallas as pl

_BLOCK = 512


def _router_kernel(x_ref, w1_ref, w2_ref, w_ref, e_ref, l_ref):
    x = x_ref[...]
    h = jax.lax.dot_general(
        x, w1_ref[...], (((1,), (1,)), ((), ())),
        preferred_element_type=jnp.float32)
    h = h * (1.0 / (1.0 + jnp.exp(-h)))
    logits = jax.lax.dot_general(
        h, w2_ref[...], (((1,), (1,)), ((), ())),
        preferred_element_type=jnp.float32)
    l_ref[...] = logits

    n_exp = logits.shape[1]
    iota = jax.lax.broadcasted_iota(jnp.int32, logits.shape, 1)
    m1 = jnp.max(logits, axis=1, keepdims=True)
    a1 = jnp.min(jnp.where(logits == m1, iota, n_exp), axis=1, keepdims=True)
    masked = jnp.where(iota == a1, -jnp.inf, logits)
    m2 = jnp.max(masked, axis=1, keepdims=True)
    a2 = jnp.min(jnp.where(masked == m2, iota, n_exp), axis=1, keepdims=True)
    e = jnp.exp(m2 - m1)
    s = 1.0 + e
    w_ref[...] = jnp.concatenate([1.0 / s, e / s], axis=1)
    e_ref[...] = jnp.concatenate([a1, a2], axis=1)


def kernel(x, W1, W2):
    n_tokens, hidden = x.shape
    n_exp = W2.shape[0]
    block = min(_BLOCK, n_tokens)
    grid = (n_tokens // block,)
    weights, experts, logits = pl.pallas_call(
        _router_kernel,
        grid=grid,
        in_specs=[
            pl.BlockSpec((block, hidden), lambda i: (i, 0)),
            pl.BlockSpec((hidden, hidden), lambda i: (0, 0)),
            pl.BlockSpec((n_exp, hidden), lambda i: (0, 0)),
        ],
        out_specs=[
            pl.BlockSpec((block, 2), lambda i: (i, 0)),
            pl.BlockSpec((block, 2), lambda i: (i, 0)),
            pl.BlockSpec((block, n_exp), lambda i: (i, 0)),
        ],
        out_shape=[
            jax.ShapeDtypeStruct((n_tokens, 2), jnp.float32),
            jax.ShapeDtypeStruct((n_tokens, 2), jnp.int32),
            jax.ShapeDtypeStruct((n_tokens, n_exp), jnp.float32),
        ],
    )(x, W1, W2)
    return weights, experts, logits


# speedup vs baseline: 1.0018x; 1.0018x over previous
import jax
import jax.numpy as jnp
from jax.experimental import pallas as pl
from jax.experimental.pallas import tpu as pltpu

_BLOCK = 512


def _router_kernel(x_ref, w1_ref, w2_ref, w_ref, e_ref, l_ref):
    x = x_ref[...]
    h = jax.lax.dot_general(
        x, w1_ref[...], (((1,), (1,)), ((), ())),
        preferred_element_type=jnp.float32)
    h = h * (1.0 / (1.0 + jnp.exp(-h)))
    logits = jax.lax.dot_general(
        h, w2_ref[...], (((1,), (1,)), ((), ())),
        preferred_element_type=jnp.float32)
    l_ref[...] = logits

    n_exp = logits.shape[1]
    iota = jax.lax.broadcasted_iota(jnp.int32, logits.shape, 1)
    m1 = jnp.max(logits, axis=1, keepdims=True)
    a1 = jnp.min(jnp.where(logits == m1, iota, n_exp), axis=1, keepdims=True)
    masked = jnp.where(iota == a1, -jnp.inf, logits)
    m2 = jnp.max(masked, axis=1, keepdims=True)
    a2 = jnp.min(jnp.where(masked == m2, iota, n_exp), axis=1, keepdims=True)
    e = jnp.exp(m2 - m1)
    s = 1.0 + e
    w_ref[...] = jnp.concatenate([1.0 / s, e / s], axis=1)
    e_ref[...] = jnp.concatenate([a1, a2], axis=1)


def kernel(x, W1, W2):
    n_tokens, hidden = x.shape
    n_exp = W2.shape[0]
    block = min(_BLOCK, n_tokens)
    grid = (n_tokens // block,)
    weights, experts, logits = pl.pallas_call(
        _router_kernel,
        grid=grid,
        in_specs=[
            pl.BlockSpec((block, hidden), lambda i: (i, 0)),
            pl.BlockSpec((hidden, hidden), lambda i: (0, 0)),
            pl.BlockSpec((n_exp, hidden), lambda i: (0, 0)),
        ],
        out_specs=[
            pl.BlockSpec((block, 2), lambda i: (i, 0)),
            pl.BlockSpec((block, 2), lambda i: (i, 0)),
            pl.BlockSpec((block, n_exp), lambda i: (i, 0)),
        ],
        out_shape=[
            jax.ShapeDtypeStruct((n_tokens, 2), jnp.float32),
            jax.ShapeDtypeStruct((n_tokens, 2), jnp.int32),
            jax.ShapeDtypeStruct((n_tokens, n_exp), jnp.float32),
        ],
        compiler_params=pltpu.CompilerParams(
            dimension_semantics=("parallel",)),
    )(x, W1, W2)
    return weights, experts, logits


# speedup vs baseline: 1.0621x; 1.0602x over previous
import jax
import jax.numpy as jnp
from jax.experimental import pallas as pl
from jax.experimental.pallas import tpu as pltpu

_BLOCK = 1024


def _router_kernel(x_ref, w1_ref, w2_ref, w_ref, e_ref, l_ref):
    x = x_ref[...]
    h = jax.lax.dot_general(
        x, w1_ref[...], (((1,), (1,)), ((), ())),
        preferred_element_type=jnp.float32)
    h = h * (1.0 / (1.0 + jnp.exp(-h)))
    logits = jax.lax.dot_general(
        h, w2_ref[...], (((1,), (1,)), ((), ())),
        preferred_element_type=jnp.float32)
    l_ref[...] = logits

    n_exp = logits.shape[1]
    iota = jax.lax.broadcasted_iota(jnp.int32, logits.shape, 1)
    m1 = jnp.max(logits, axis=1, keepdims=True)
    a1 = jnp.min(jnp.where(logits == m1, iota, n_exp), axis=1, keepdims=True)
    masked = jnp.where(iota == a1, -jnp.inf, logits)
    m2 = jnp.max(masked, axis=1, keepdims=True)
    a2 = jnp.min(jnp.where(masked == m2, iota, n_exp), axis=1, keepdims=True)
    e = jnp.exp(m2 - m1)
    s = 1.0 + e
    w_ref[...] = jnp.concatenate([1.0 / s, e / s], axis=1)
    e_ref[...] = jnp.concatenate([a1, a2], axis=1)


def kernel(x, W1, W2):
    n_tokens, hidden = x.shape
    n_exp = W2.shape[0]
    block = min(_BLOCK, n_tokens)
    grid = (n_tokens // block,)
    weights, experts, logits = pl.pallas_call(
        _router_kernel,
        grid=grid,
        in_specs=[
            pl.BlockSpec((block, hidden), lambda i: (i, 0)),
            pl.BlockSpec((hidden, hidden), lambda i: (0, 0)),
            pl.BlockSpec((n_exp, hidden), lambda i: (0, 0)),
        ],
        out_specs=[
            pl.BlockSpec((block, 2), lambda i: (i, 0)),
            pl.BlockSpec((block, 2), lambda i: (i, 0)),
            pl.BlockSpec((block, n_exp), lambda i: (i, 0)),
        ],
        out_shape=[
            jax.ShapeDtypeStruct((n_tokens, 2), jnp.float32),
            jax.ShapeDtypeStruct((n_tokens, 2), jnp.int32),
            jax.ShapeDtypeStruct((n_tokens, n_exp), jnp.float32),
        ],
        compiler_params=pltpu.CompilerParams(
            dimension_semantics=("parallel",)),
    )(x, W1, W2)
    return weights, experts, logits
